# baseline (speedup 1.0000x reference)
import jax
import jax.numpy as jnp
from jax import lax
from jax.experimental import pallas as pl
from jax.experimental.pallas import tpu as pltpu
from jax.experimental.pallas import tpu_sc as plsc

N_CLASSES = 1000
D = 128
B = 16384
ALPHA = 0.5

NC = 2
NS = 16
L = 16
NW = NC * NS
ROWS_W = B // NW
CHUNK = 128
NCHUNK = ROWS_W // CHUNK
CPAD = 1024
CROWS = CPAD // NS
CW = 128


def _sc_body(feat_hbm, lab_hbm, out_sums, out_counts, out_ssq,
             feat_v, lab_v, ones_v, zs_v, zc_v, cb_v, ssq_v,
             stage_sems, scat_sem, zero_sem,
             sums_sh, counts_sh):
    c = lax.axis_index("c")
    s = lax.axis_index("s")
    w = c * NS + s
    base = w * ROWS_W
    rowbase = s * CROWS

    stage = [
        pltpu.async_copy(feat_hbm.at[pl.ds(base + j * CHUNK, CHUNK)],
                         feat_v.at[pl.ds(j * CHUNK, CHUNK)],
                         stage_sems.at[j])
        for j in range(NCHUNK)
    ]
    pltpu.sync_copy(lab_hbm.at[pl.ds(w * NCHUNK, NCHUNK)], lab_v)

    zvec = jnp.zeros((L,), jnp.float32)
    ovec = jnp.ones((L,), jnp.float32)

    def fill_zs(r, _):
        for jj in range(D // L):
            zs_v[r, pl.ds(jj * L, L)] = zvec
        return 0
    lax.fori_loop(0, CROWS, fill_zs, 0)

    def fill_ones(r, _):
        for jj in range(CW // L):
            ones_v[r, pl.ds(jj * L, L)] = ovec
        return 0
    lax.fori_loop(0, CHUNK, fill_ones, 0)

    z0 = pltpu.async_copy(zs_v, sums_sh.at[pl.ds(rowbase, CROWS)], zero_sem)
    z1 = pltpu.async_copy(zs_v, counts_sh.at[pl.ds(rowbase, CROWS)], zero_sem)
    z0.wait()
    z1.wait()
    plsc.subcore_barrier()

    scats = []
    acc = jnp.zeros((L,), jnp.float32)
    for j in range(NCHUNK):
        stage[j].wait()
        scats.append(pltpu.async_copy(feat_v.at[pl.ds(j * CHUNK, CHUNK)],
                                      sums_sh.at[lab_v.at[j]],
                                      scat_sem, add=True))
        scats.append(pltpu.async_copy(ones_v, counts_sh.at[lab_v.at[j]],
                                      scat_sem, add=True))

        def ssq_row(r, a):
            for jj in range(D // L):
                x = feat_v[r, pl.ds(jj * L, L)]
                a = a + x * x
            return a
        acc = lax.fori_loop(j * CHUNK, (j + 1) * CHUNK, ssq_row, acc)

    ssq_v[...] = acc
    pltpu.sync_copy(ssq_v, out_ssq.at[c, s])
    for h in scats:
        h.wait()
    plsc.subcore_barrier()

    pltpu.sync_copy(sums_sh.at[pl.ds(rowbase, CROWS)],
                    out_sums.at[c, pl.ds(rowbase, CROWS)])
    pltpu.sync_copy(counts_sh.at[pl.ds(rowbase, CROWS)], cb_v)

    def col_extract(r, _):
        zc_v[r, :] = cb_v[r, pl.ds(0, L)]
        return 0
    lax.fori_loop(0, CROWS, col_extract, 0)
    pltpu.sync_copy(zc_v, out_counts.at[c, pl.ds(rowbase, CROWS)])


_sc_call = pl.kernel(
    _sc_body,
    out_type=(
        jax.ShapeDtypeStruct((NC, CPAD, D), jnp.float32),
        jax.ShapeDtypeStruct((NC, CPAD, L), jnp.float32),
        jax.ShapeDtypeStruct((NC, NS, L), jnp.float32),
    ),
    mesh=plsc.VectorSubcoreMesh(core_axis_name="c", subcore_axis_name="s",
                                num_cores=NC, num_subcores=NS),
    scratch_types=[
        pltpu.VMEM((ROWS_W, D), jnp.float32),
        pltpu.VMEM((NCHUNK, CHUNK), jnp.int32),
        pltpu.VMEM((CHUNK, CW), jnp.float32),
        pltpu.VMEM((CROWS, D), jnp.float32),
        pltpu.VMEM((CROWS, L), jnp.float32),
        pltpu.VMEM((CROWS, CW), jnp.float32),
        pltpu.VMEM((L,), jnp.float32),
        pltpu.SemaphoreType.DMA((NCHUNK,)),
        pltpu.SemaphoreType.DMA,
        pltpu.SemaphoreType.DMA,
        pltpu.VMEM_SHARED((CPAD, D), jnp.float32),
        pltpu.VMEM_SHARED((CPAD, CW), jnp.float32),
    ],
    name="center_loss_sc_segment_sum",
)


def _tc_body(psums, pcounts, pssq, cen_ref, loss_ref, newc_ref):
    sums = (psums[0] + psums[1])[:N_CLASSES]
    cnt = (pcounts[0] + pcounts[1])[:N_CLASSES, 0:1]
    cen = cen_ref[...]
    means = sums / jnp.maximum(cnt, 1.0)
    newc_ref[...] = jnp.where(cnt > 0.0, (1.0 - ALPHA) * cen + ALPHA * means, cen)
    ssq = jnp.sum(pssq[...])
    dot_sc = jnp.sum(sums * cen)
    cn = jnp.sum(cen * cen, axis=1, keepdims=True)
    loss = (ssq - 2.0 * dot_sc + jnp.sum(cnt * cn)) / (B * D)
    loss_ref[...] = jnp.broadcast_to(loss, (1, 1))


_tc_call = pl.pallas_call(
    _tc_body,
    out_shape=(
        jax.ShapeDtypeStruct((1, 1), jnp.float32),
        jax.ShapeDtypeStruct((N_CLASSES, D), jnp.float32),
    ),
)


@jax.jit
def kernel(features, labels, centers):
    lab2 = labels.reshape(NW * NCHUNK, CHUNK)
    psums, pcounts, pssq = _sc_call(features, lab2)
    loss, newc = _tc_call(psums, pcounts, pssq, centers)
    return loss[0, 0], newc


# speedup vs baseline: 1.0304x; 1.0304x over previous
import jax
import jax.numpy as jnp
from jax import lax
from jax.experimental import pallas as pl
from jax.experimental.pallas import tpu as pltpu
from jax.experimental.pallas import tpu_sc as plsc

N_CLASSES = 1000
D = 128
B = 16384
ALPHA = 0.5

NC = 2
NS = 16
L = 16
NW = NC * NS
ROWS_W = B // NW
CHUNK = 64
NCHUNK = ROWS_W // CHUNK
CPAD = 1024
CROWS = CPAD // NS
CW = 128


def _sc_body(feat_hbm, lab_hbm, out_sums, out_counts, out_ssq,
             feat_v, lab_v, ones_v, zs_v, zc_v, cb_v, ssq_v,
             stage_sems, scat_sem, zero_sem,
             sums_sh, counts_sh):
    c = lax.axis_index("c")
    s = lax.axis_index("s")
    w = c * NS + s
    base = w * ROWS_W
    rowbase = s * CROWS

    stage = [
        pltpu.async_copy(feat_hbm.at[pl.ds(base + j * CHUNK, CHUNK)],
                         feat_v.at[pl.ds(j * CHUNK, CHUNK)],
                         stage_sems.at[j])
        for j in range(NCHUNK)
    ]
    pltpu.sync_copy(lab_hbm.at[pl.ds(w * NCHUNK, NCHUNK)], lab_v)

    zvec = jnp.zeros((L,), jnp.float32)
    ovec = jnp.ones((L,), jnp.float32)

    def fill_zs(r, _):
        for jj in range(D // L):
            zs_v[r, pl.ds(jj * L, L)] = zvec
        return 0
    lax.fori_loop(0, CROWS, fill_zs, 0)

    def fill_ones(r, _):
        for jj in range(CW // L):
            ones_v[r, pl.ds(jj * L, L)] = ovec
        return 0
    lax.fori_loop(0, CHUNK, fill_ones, 0)

    z0 = pltpu.async_copy(zs_v, sums_sh.at[pl.ds(rowbase, CROWS)], zero_sem)
    z1 = pltpu.async_copy(zs_v, counts_sh.at[pl.ds(rowbase, CROWS)], zero_sem)
    z0.wait()
    z1.wait()
    plsc.subcore_barrier()

    scats = []
    acc = jnp.zeros((L,), jnp.float32)
    for j in range(NCHUNK):
        stage[j].wait()
        scats.append(pltpu.async_copy(feat_v.at[pl.ds(j * CHUNK, CHUNK)],
                                      sums_sh.at[lab_v.at[j]],
                                      scat_sem, add=True))
        scats.append(pltpu.async_copy(ones_v, counts_sh.at[lab_v.at[j]],
                                      scat_sem, add=True))

        def ssq_row(r, a):
            for jj in range(D // L):
                x = feat_v[r, pl.ds(jj * L, L)]
                a = a + x * x
            return a
        acc = lax.fori_loop(j * CHUNK, (j + 1) * CHUNK, ssq_row, acc)

    ssq_v[...] = acc
    pltpu.sync_copy(ssq_v, out_ssq.at[c, s])
    for h in scats:
        h.wait()
    plsc.subcore_barrier()

    pltpu.sync_copy(sums_sh.at[pl.ds(rowbase, CROWS)],
                    out_sums.at[c, pl.ds(rowbase, CROWS)])
    pltpu.sync_copy(counts_sh.at[pl.ds(rowbase, CROWS)], cb_v)

    def col_extract(r, _):
        zc_v[r, :] = cb_v[r, pl.ds(0, L)]
        return 0
    lax.fori_loop(0, CROWS, col_extract, 0)
    pltpu.sync_copy(zc_v, out_counts.at[c, pl.ds(rowbase, CROWS)])


_sc_call = pl.kernel(
    _sc_body,
    out_type=(
        jax.ShapeDtypeStruct((NC, CPAD, D), jnp.float32),
        jax.ShapeDtypeStruct((NC, CPAD, L), jnp.float32),
        jax.ShapeDtypeStruct((NC, NS, L), jnp.float32),
    ),
    mesh=plsc.VectorSubcoreMesh(core_axis_name="c", subcore_axis_name="s",
                                num_cores=NC, num_subcores=NS),
    scratch_types=[
        pltpu.VMEM((ROWS_W, D), jnp.float32),
        pltpu.VMEM((NCHUNK, CHUNK), jnp.int32),
        pltpu.VMEM((CHUNK, CW), jnp.float32),
        pltpu.VMEM((CROWS, D), jnp.float32),
        pltpu.VMEM((CROWS, L), jnp.float32),
        pltpu.VMEM((CROWS, CW), jnp.float32),
        pltpu.VMEM((L,), jnp.float32),
        pltpu.SemaphoreType.DMA((NCHUNK,)),
        pltpu.SemaphoreType.DMA,
        pltpu.SemaphoreType.DMA,
        pltpu.VMEM_SHARED((CPAD, D), jnp.float32),
        pltpu.VMEM_SHARED((CPAD, CW), jnp.float32),
    ],
    name="center_loss_sc_segment_sum",
)


def _tc_body(psums, pcounts, pssq, cen_ref, loss_ref, newc_ref):
    sums = (psums[0] + psums[1])[:N_CLASSES]
    cnt = (pcounts[0] + pcounts[1])[:N_CLASSES, 0:1]
    cen = cen_ref[...]
    means = sums / jnp.maximum(cnt, 1.0)
    newc_ref[...] = jnp.where(cnt > 0.0, (1.0 - ALPHA) * cen + ALPHA * means, cen)
    ssq = jnp.sum(pssq[...])
    dot_sc = jnp.sum(sums * cen)
    cn = jnp.sum(cen * cen, axis=1, keepdims=True)
    loss = (ssq - 2.0 * dot_sc + jnp.sum(cnt * cn)) / (B * D)
    loss_ref[...] = jnp.broadcast_to(loss, (1, 1))


_tc_call = pl.pallas_call(
    _tc_body,
    out_shape=(
        jax.ShapeDtypeStruct((1, 1), jnp.float32),
        jax.ShapeDtypeStruct((N_CLASSES, D), jnp.float32),
    ),
)


@jax.jit
def kernel(features, labels, centers):
    lab2 = labels.reshape(NW * NCHUNK, CHUNK)
    psums, pcounts, pssq = _sc_call(features, lab2)
    loss, newc = _tc_call(psums, pcounts, pssq, centers)
    return loss[0, 0], newc


# speedup vs baseline: 1.0865x; 1.0545x over previous
import jax
import jax.numpy as jnp
from jax import lax
from jax.experimental import pallas as pl
from jax.experimental.pallas import tpu as pltpu
from jax.experimental.pallas import tpu_sc as plsc

N_CLASSES = 1000
D = 128
B = 16384
ALPHA = 0.5

NC = 2
NS = 16
L = 16
NW = NC * NS
ROWS_W = B // NW
CHUNK = 128
NCHUNK = ROWS_W // CHUNK
CPAD = 1024
CROWS = CPAD // NS
CW = 128


def _sc_body(feat_hbm, lab_hbm, out_sums, out_counts, out_ssq,
             feat_v, lab_v, ones_v, zs_v, zc_v, cb_v, ssq_v,
             stage_sems, scat_sem, zero_sem,
             sums_sh, counts_sh):
    c = lax.axis_index("c")
    s = lax.axis_index("s")
    w = c * NS + s
    base = w * ROWS_W
    rowbase = s * CROWS

    stage = [
        pltpu.async_copy(feat_hbm.at[pl.ds(base + j * CHUNK, CHUNK)],
                         feat_v.at[pl.ds(j * CHUNK, CHUNK)],
                         stage_sems.at[j])
        for j in range(NCHUNK)
    ]
    pltpu.sync_copy(lab_hbm.at[pl.ds(w * NCHUNK, NCHUNK)], lab_v)

    zvec = jnp.zeros((L,), jnp.float32)
    ovec = jnp.ones((L,), jnp.float32)

    def fill_zs(r, _):
        for jj in range(D // L):
            zs_v[r, pl.ds(jj * L, L)] = zvec
        return 0
    lax.fori_loop(0, CROWS, fill_zs, 0)

    def fill_ones(r, _):
        for jj in range(CW // L):
            ones_v[r, pl.ds(jj * L, L)] = ovec
        return 0
    lax.fori_loop(0, CHUNK, fill_ones, 0)

    z0 = pltpu.async_copy(zs_v, sums_sh.at[pl.ds(rowbase, CROWS)], zero_sem)
    z1 = pltpu.async_copy(zs_v, counts_sh.at[pl.ds(rowbase, CROWS)], zero_sem)
    z0.wait()
    z1.wait()
    plsc.subcore_barrier()

    scats = []
    acc = jnp.zeros((L,), jnp.float32)
    for j in range(NCHUNK):
        stage[j].wait()
        scats.append(pltpu.async_copy(feat_v.at[pl.ds(j * CHUNK, CHUNK)],
                                      sums_sh.at[lab_v.at[j]],
                                      scat_sem, add=True))
        scats.append(pltpu.async_copy(ones_v, counts_sh.at[lab_v.at[j]],
                                      scat_sem, add=True))

        def ssq_row(r, a):
            for jj in range(D // L):
                x = feat_v[r, pl.ds(jj * L, L)]
                a = a + x * x
            return a
        acc = lax.fori_loop(j * CHUNK, (j + 1) * CHUNK, ssq_row, acc)

    ssq_v[...] = acc
    pltpu.sync_copy(ssq_v, out_ssq.at[c, s])
    for h in scats:
        h.wait()
    plsc.subcore_barrier()

    pltpu.sync_copy(sums_sh.at[pl.ds(rowbase, CROWS)],
                    out_sums.at[c, pl.ds(rowbase, CROWS)])
    pltpu.sync_copy(counts_sh.at[pl.ds(rowbase, CROWS)], cb_v)

    def col_extract(r, _):
        zc_v[r, :] = cb_v[r, pl.ds(0, L)]
        return 0
    lax.fori_loop(0, CROWS, col_extract, 0)
    pltpu.sync_copy(zc_v, out_counts.at[c, pl.ds(rowbase, CROWS)])


_sc_call = pl.kernel(
    _sc_body,
    out_type=(
        jax.ShapeDtypeStruct((NC, CPAD, D), jnp.float32),
        jax.ShapeDtypeStruct((NC, CPAD, L), jnp.float32),
        jax.ShapeDtypeStruct((NC, NS, L), jnp.float32),
    ),
    mesh=plsc.VectorSubcoreMesh(core_axis_name="c", subcore_axis_name="s",
                                num_cores=NC, num_subcores=NS),
    scratch_types=[
        pltpu.VMEM((ROWS_W, D), jnp.float32),
        pltpu.VMEM((NCHUNK, CHUNK), jnp.int32),
        pltpu.VMEM((CHUNK, CW), jnp.float32),
        pltpu.VMEM((CROWS, D), jnp.float32),
        pltpu.VMEM((CROWS, L), jnp.float32),
        pltpu.VMEM((CROWS, CW), jnp.float32),
        pltpu.VMEM((L,), jnp.float32),
        pltpu.SemaphoreType.DMA((NCHUNK,)),
        pltpu.SemaphoreType.DMA,
        pltpu.SemaphoreType.DMA,
        pltpu.VMEM_SHARED((CPAD, D), jnp.float32),
        pltpu.VMEM_SHARED((CPAD, CW), jnp.float32),
    ],
    name="center_loss_sc_segment_sum",
)


def _tc_body(psums, pcounts, pssq, cen_ref, loss_ref, newc_ref):
    sums = (psums[0] + psums[1])[:N_CLASSES]
    cnt = (pcounts[0] + pcounts[1])[:N_CLASSES, 0:1]
    cen = cen_ref[...]
    means = sums / jnp.maximum(cnt, 1.0)
    newc_ref[...] = jnp.where(cnt > 0.0, (1.0 - ALPHA) * cen + ALPHA * means, cen)
    ssq = jnp.sum(pssq[...])
    dot_sc = jnp.sum(sums * cen)
    cn = jnp.sum(cen * cen, axis=1, keepdims=True)
    loss = (ssq - 2.0 * dot_sc + jnp.sum(cnt * cn)) / (B * D)
    loss_ref[...] = jnp.broadcast_to(loss, (1, 1))


_tc_call = pl.pallas_call(
    _tc_body,
    out_shape=(
        jax.ShapeDtypeStruct((1, 1), jnp.float32),
        jax.ShapeDtypeStruct((N_CLASSES, D), jnp.float32),
    ),
)


@jax.jit
def kernel(features, labels, centers):
    lab2 = labels.reshape(NW * NCHUNK, CHUNK)
    psums, pcounts, pssq = _sc_call(features, lab2)
    loss, newc = _tc_call(psums, pcounts, pssq, centers)
    return loss[0, 0], newc
